# baseline (speedup 1.0000x reference)
import functools

import jax
import jax.numpy as jnp
from jax import lax
from jax.experimental import pallas as pl
from jax.experimental.pallas import tpu as pltpu
from jax.experimental.pallas import tpu_sc as plsc

V = 1000000
D = 64
B = 16384
WIN = 512
NWIN = V // WIN
TAIL_START = NWIN * WIN
N_IT = 62
BKT_CAP = B + 16 * N_IT
OBROWS = 64
TRASH = B
OUT2_ROWS = B + 8


def _reset_pos(pos_v):
    for kk in range(OBROWS // 16):
        pos_v[pl.ds(kk * 16, 16)] = jnp.broadcast_to(jnp.int32(TRASH), (16,))


def _flush_if(cond_val, ob_v, pos_v, out_hbm, sem_o, s_ob):
    def flush(sf):
        pltpu.async_copy(ob_v, out_hbm.at[pos_v], sem_o).wait()
        _reset_pos(pos_v)
        return jnp.int32(0)

    return lax.cond(cond_val, flush, lambda sf: sf, s_ob)


def _process_window(buf, it, lo, iota, bkt_idx, bkt_pos, cnt_v, off_v,
                    ob_v, pos_v, out_hbm, sem_o, s_ob):
    it_s = jnp.broadcast_to(it, (16,))
    cw = plsc.load_gather(cnt_v, [it_s])
    ow = plsc.load_gather(off_v, [it_s])
    cs = lax.reduce_max(cw, axes=(0,))
    os_ = lax.reduce_max(ow, axes=(0,))
    lo_v = jnp.broadcast_to(lo, (16,))
    n_c = (cs + 15) >> 4

    def vbody(c, s):
        s = _flush_if(s + 16 > OBROWS, ob_v, pos_v, out_hbm, sem_o, s)
        base = os_ + c * 16
        hv = bkt_idx[pl.ds(base, 16)]
        pv = bkt_pos[pl.ds(base, 16)]
        m = (c * 16 + iota) < cw
        col_s = jnp.where(m, hv - lo_v, 0)
        rows = jnp.broadcast_to(s, (16,)) + iota
        for f in range(D):
            f_s = jnp.full((16,), f, jnp.int32)
            vals = plsc.load_gather(buf, [f_s, col_s])
            plsc.store_scatter(ob_v, [rows, f_s], vals, mask=m)
        plsc.store_scatter(pos_v, [rows], pv, mask=m)
        return s + jnp.clip(cs - c * 16, 0, 16)

    return lax.fori_loop(0, n_c, vbody, s_ob)


@functools.cache
def _make_sc_gather():
    info = plsc.get_sparse_core_info()
    NC, NS = info.num_cores, info.num_subcores
    NW = NC * NS
    mesh = plsc.VectorSubcoreMesh(core_axis_name="c", subcore_axis_name="s")

    @functools.partial(
        pl.kernel,
        mesh=mesh,
        compiler_params=pltpu.CompilerParams(needs_layout_passes=False),
        out_type=jax.ShapeDtypeStruct((OUT2_ROWS, 128), jnp.float32),
        scratch_types=[
            pltpu.VMEM((B,), jnp.int32),
            pltpu.VMEM((BKT_CAP,), jnp.int32),
            pltpu.VMEM((BKT_CAP,), jnp.int32),
            pltpu.VMEM((64,), jnp.int32),
            pltpu.VMEM((64,), jnp.int32),
            pltpu.VMEM((64,), jnp.int32),
            pltpu.VMEM((D, WIN), jnp.float32),
            pltpu.VMEM((D, WIN), jnp.float32),
            pltpu.VMEM((OBROWS, 128), jnp.float32),
            pltpu.VMEM((OBROWS,), jnp.int32),
            pltpu.SemaphoreType.DMA,
            pltpu.SemaphoreType.DMA,
            pltpu.SemaphoreType.DMA,
        ],
    )
    def gather(idx_hbm, tab_t_hbm, tail_t_hbm, out_hbm,
               idx_v, bkt_idx, bkt_pos, cnt_v, off_v, fill_v, win0, win1,
               ob_v, pos_v, sem0, sem1, sem_o):
        wid = lax.axis_index("s") * NC + lax.axis_index("c")
        iota = lax.iota(jnp.int32, 16)
        pltpu.sync_copy(idx_hbm, idx_v)
        _reset_pos(pos_v)
        for kk in range(4):
            cnt_v[pl.ds(kk * 16, 16)] = jnp.broadcast_to(jnp.int32(0), (16,))

        def start_dma(it, buf, sem):
            w_id = wid + NW * it

            @pl.when(w_id < NWIN)
            def _():
                pltpu.async_copy(
                    tab_t_hbm.at[:, pl.ds(pl.multiple_of(w_id * WIN, 128),
                                          WIN)],
                    buf, sem)

        def wait_dma(it, buf, sem):
            w_id = wid + NW * it

            @pl.when(w_id < NWIN)
            def _():
                pltpu.make_async_copy(
                    tab_t_hbm.at[:, pl.ds(0, WIN)], buf, sem).wait()

            @pl.when(w_id == NWIN)
            def _():
                pltpu.sync_copy(tail_t_hbm, buf.at[:, pl.ds(0, 128)])

        start_dma(jnp.int32(0), win0, sem0)

        def h_body(v, s):
            idxv = idx_v[pl.ds(v * 16, 16)]
            m = (jnp.right_shift(idxv, 9) & (NW - 1)) == wid
            w_loc = jnp.right_shift(idxv, 14)
            cnt1, last = plsc.scan_count(w_loc, mask=m)
            plsc.addupdate_scatter(cnt_v, [w_loc], cnt1, mask=m & last)
            return s

        lax.fori_loop(0, B // 16, h_body, jnp.int32(0))

        running = jnp.int32(0)
        for kk in range(4):
            cv = cnt_v[pl.ds(kk * 16, 16)]
            cva = (cv + 15) & ~jnp.int32(15)
            excl = plsc.cumsum(cva) - cva + jnp.broadcast_to(running, (16,))
            off_v[pl.ds(kk * 16, 16)] = excl
            fill_v[pl.ds(kk * 16, 16)] = excl
            running = running + lax.reduce_sum(cva, axes=(0,))

        def f_body(v, s):
            idxv = idx_v[pl.ds(v * 16, 16)]
            m = (jnp.right_shift(idxv, 9) & (NW - 1)) == wid
            w_loc = jnp.right_shift(idxv, 14)
            cnt1, last = plsc.scan_count(w_loc, mask=m)
            basev = plsc.load_gather(fill_v, [w_loc])
            slot = basev + cnt1 - 1
            plsc.store_scatter(bkt_idx, [slot], idxv, mask=m)
            plsc.store_scatter(bkt_pos, [slot], v * 16 + iota, mask=m)
            plsc.addupdate_scatter(fill_v, [w_loc], cnt1, mask=m & last)
            return s

        lax.fori_loop(0, B // 16, f_body, jnp.int32(0))

        def it_body(it, s_ob):
            w_id = wid + NW * it
            lo = w_id * WIN

            def with_buf(buf, sem, s_ob):
                wait_dma(it, buf, sem)

                def proc(s):
                    return _process_window(
                        buf, it, lo, iota, bkt_idx, bkt_pos, cnt_v, off_v,
                        ob_v, pos_v, out_hbm, sem_o, s)

                return lax.cond(w_id <= NWIN, proc, lambda s: s, s_ob)

            def even(s):
                start_dma(it + 1, win1, sem1)
                return with_buf(win0, sem0, s)

            def odd(s):
                start_dma(it + 1, win0, sem0)
                return with_buf(win1, sem1, s)

            return lax.cond((it & 1) == 0, even, odd, s_ob)

        s_ob = lax.fori_loop(0, N_IT, it_body, jnp.int32(0))

        @pl.when(s_ob > 0)
        def _():
            pltpu.async_copy(ob_v, out_hbm.at[pos_v], sem_o).wait()

    return gather


def _silu_linear_t_body(x2_ref, w_ref, b_ref, o_ref):
    x = x2_ref[:, :D]
    s = x / (1.0 + jnp.exp(-x))
    o_ref[...] = (
        lax.dot_general(w_ref[...], s, (((1,), (1,)), ((), ())),
                        preferred_element_type=jnp.float32)
        + b_ref[...]
    )


@functools.cache
def _make_tc_silu_linear_t(O, blk):
    return pl.pallas_call(
        _silu_linear_t_body,
        grid=(B // blk,),
        in_specs=[
            pl.BlockSpec((blk, 128), lambda i: (i, 0)),
            pl.BlockSpec((O, D), lambda i: (0, 0)),
            pl.BlockSpec((O, 1), lambda i: (0, 0)),
        ],
        out_specs=pl.BlockSpec((O, blk), lambda i: (0, i)),
        out_shape=jax.ShapeDtypeStruct((O, B), jnp.float32),
    )


def kernel(input, emb_table, W, b):
    O = W.shape[0]
    idx = input.astype(jnp.int32)
    tab_t = emb_table.T
    tail_t = jnp.pad(
        lax.slice(emb_table, (TAIL_START, 0), (V, D)).T,
        ((0, 0), (0, 128 - (V - TAIL_START))))
    x2 = _make_sc_gather()(idx, tab_t, tail_t)
    out_t = _make_tc_silu_linear_t(O, 2048)(x2, W, b.reshape(O, 1))
    return out_t.T


# speedup vs baseline: 2.1955x; 2.1955x over previous
import functools

import jax
import jax.numpy as jnp
from jax import lax
from jax.experimental import pallas as pl
from jax.experimental.pallas import tpu as pltpu
from jax.experimental.pallas import tpu_sc as plsc

V = 1000000
D = 64
B = 16384
WIN = 512
NWIN = V // WIN
TAIL_START = NWIN * WIN
N_IT = 62
HITCAP = 4096
OBROWS = 64
TRASH = B
OUT2_ROWS = B + 8


def _reset_pos(pos_v):
    for kk in range(OBROWS // 16):
        pos_v[pl.ds(kk * 16, 16)] = jnp.broadcast_to(jnp.int32(TRASH), (16,))


def _flush_if(cond_val, ob_v, pos_v, out_hbm, sem_o, s_ob):
    def flush(sf):
        pltpu.async_copy(ob_v, out_hbm.at[pos_v], sem_o).wait()
        _reset_pos(pos_v)
        return jnp.int32(0)

    return lax.cond(cond_val, flush, lambda sf: sf, s_ob)


def _extract_hits(buf, lo, iota, hv, pv, m, ob_v, pos_v, out_hbm, sem_o,
                  s_ob):
    m_int0 = lax.reduce_sum(
        jnp.where(m, jnp.left_shift(jnp.int32(1), iota), 0), axes=(0,)
    )

    def cond(c):
        return c[0] != 0

    def body(c):
        m_int, s = c
        low = m_int & (-m_int)
        lane_m = (jnp.right_shift(jnp.broadcast_to(low, (16,)), iota) & 1) == 1
        col = lax.reduce_sum(jnp.where(lane_m, hv, 0), axes=(0,)) - lo
        p = lax.reduce_sum(jnp.where(lane_m, pv, 0), axes=(0,))
        col_s = jnp.broadcast_to(col, (16,))
        row_s = jnp.broadcast_to(s, (16,))
        for k in range(4):
            val = plsc.load_gather(buf, [iota + 16 * k, col_s])
            plsc.store_scatter(ob_v, [row_s, iota + 16 * k], val)
        plsc.store_scatter(pos_v, [row_s], jnp.broadcast_to(p, (16,)),
                           mask=iota == 0)
        s = _flush_if(s + 1 == OBROWS, ob_v, pos_v, out_hbm, sem_o, s + 1)
        return m_int & (m_int - 1), s

    _, s_ob = lax.while_loop(cond, body, (m_int0, s_ob))
    return s_ob


def _process_window(buf, lo, hi, iota, fast, s_scan, idx_v, hit_idx, hit_pos,
                    ob_v, pos_v, out_hbm, sem_o, s_ob):
    if fast:
        n_c = (s_scan + 15) >> 4

        def cbody(c, s):
            base = c * 16
            hv = hit_idx[pl.ds(base, 16)]
            pv = hit_pos[pl.ds(base, 16)]
            m = (hv >= lo) & (hv < hi) & ((base + iota) < s_scan)
            return _extract_hits(buf, lo, iota, hv, pv, m, ob_v, pos_v,
                                 out_hbm, sem_o, s)

        return lax.fori_loop(0, n_c, cbody, s_ob)
    else:

        def cbody(c, s):
            hv = idx_v[pl.ds(c * 16, 16)]
            pv = c * 16 + iota
            m = (hv >= lo) & (hv < hi)
            return _extract_hits(buf, lo, iota, hv, pv, m, ob_v, pos_v,
                                 out_hbm, sem_o, s)

        return lax.fori_loop(0, B // 16, cbody, s_ob)


@functools.cache
def _make_sc_gather():
    info = plsc.get_sparse_core_info()
    NC, NS = info.num_cores, info.num_subcores
    NW = NC * NS
    mesh = plsc.VectorSubcoreMesh(core_axis_name="c", subcore_axis_name="s")

    @functools.partial(
        pl.kernel,
        mesh=mesh,
        compiler_params=pltpu.CompilerParams(needs_layout_passes=False),
        out_type=jax.ShapeDtypeStruct((OUT2_ROWS, 128), jnp.float32),
        scratch_types=[
            pltpu.VMEM((B,), jnp.int32),
            pltpu.VMEM((HITCAP,), jnp.int32),
            pltpu.VMEM((HITCAP,), jnp.int32),
            pltpu.VMEM((D, WIN), jnp.float32),
            pltpu.VMEM((D, WIN), jnp.float32),
            pltpu.VMEM((OBROWS, 128), jnp.float32),
            pltpu.VMEM((OBROWS,), jnp.int32),
            pltpu.SemaphoreType.DMA,
            pltpu.SemaphoreType.DMA,
            pltpu.SemaphoreType.DMA,
        ],
    )
    def gather(idx_hbm, tab_t_hbm, tail_t_hbm, out_hbm,
               idx_v, hit_idx, hit_pos, win0, win1,
               ob_v, pos_v, sem0, sem1, sem_o):
        wid = lax.axis_index("s") * NC + lax.axis_index("c")
        iota = lax.iota(jnp.int32, 16)
        pltpu.sync_copy(idx_hbm, idx_v)
        _reset_pos(pos_v)

        def start_dma(it, buf, sem):
            w_id = wid + NW * it

            @pl.when(w_id < NWIN)
            def _():
                pltpu.async_copy(
                    tab_t_hbm.at[:, pl.ds(pl.multiple_of(w_id * WIN, 128),
                                          WIN)],
                    buf, sem)

        def wait_dma(it, buf, sem):
            w_id = wid + NW * it

            @pl.when(w_id < NWIN)
            def _():
                pltpu.make_async_copy(
                    tab_t_hbm.at[:, pl.ds(0, WIN)], buf, sem).wait()

            @pl.when(w_id == NWIN)
            def _():
                pltpu.sync_copy(tail_t_hbm, buf.at[:, pl.ds(0, 128)])

        start_dma(jnp.int32(0), win0, sem0)

        def h_body(v, s):
            idxv = idx_v[pl.ds(v * 16, 16)]
            m = (jnp.right_shift(idxv, 9) & (NW - 1)) == wid
            m1 = jnp.where(m, 1, 0)
            ranks = plsc.cumsum(m1) - 1
            slot = s + ranks
            mw = m & (slot < HITCAP)
            plsc.store_scatter(hit_idx, [slot], idxv, mask=mw)
            plsc.store_scatter(hit_pos, [slot], v * 16 + iota, mask=mw)
            return s + lax.reduce_sum(m1, axes=(0,))

        s_hits = lax.fori_loop(0, B // 16, h_body, jnp.int32(0))
        ovf = s_hits > HITCAP
        s_scan = jnp.minimum(s_hits, HITCAP)

        def it_body(it, s_ob):
            w_id = wid + NW * it
            lo = w_id * WIN

            def with_buf(buf, sem, s_ob):
                wait_dma(it, buf, sem)

                def proc_fast(s):
                    return _process_window(
                        buf, lo, lo + WIN, iota, True, s_scan, idx_v,
                        hit_idx, hit_pos, ob_v, pos_v, out_hbm, sem_o, s)

                def proc_slow(s):
                    return _process_window(
                        buf, lo, lo + WIN, iota, False, s_scan, idx_v,
                        hit_idx, hit_pos, ob_v, pos_v, out_hbm, sem_o, s)

                return lax.cond(
                    w_id <= NWIN,
                    lambda s: lax.cond(ovf, proc_slow, proc_fast, s),
                    lambda s: s, s_ob)

            def even(s):
                start_dma(it + 1, win1, sem1)
                return with_buf(win0, sem0, s)

            def odd(s):
                start_dma(it + 1, win0, sem0)
                return with_buf(win1, sem1, s)

            return lax.cond((it & 1) == 0, even, odd, s_ob)

        s_ob = lax.fori_loop(0, N_IT, it_body, jnp.int32(0))

        @pl.when(s_ob > 0)
        def _():
            pltpu.async_copy(ob_v, out_hbm.at[pos_v], sem_o).wait()

    return gather


def _silu_linear_t_body(x2_ref, w_ref, b_ref, o_ref):
    x = x2_ref[:, :D]
    s = x / (1.0 + jnp.exp(-x))
    o_ref[...] = (
        lax.dot_general(w_ref[...], s, (((1,), (1,)), ((), ())),
                        preferred_element_type=jnp.float32)
        + b_ref[...]
    )


@functools.cache
def _make_tc_silu_linear_t(O, blk):
    return pl.pallas_call(
        _silu_linear_t_body,
        grid=(B // blk,),
        in_specs=[
            pl.BlockSpec((blk, 128), lambda i: (i, 0)),
            pl.BlockSpec((O, D), lambda i: (0, 0)),
            pl.BlockSpec((O, 1), lambda i: (0, 0)),
        ],
        out_specs=pl.BlockSpec((O, blk), lambda i: (0, i)),
        out_shape=jax.ShapeDtypeStruct((O, B), jnp.float32),
    )


def kernel(input, emb_table, W, b):
    O = W.shape[0]
    idx = input.astype(jnp.int32)
    tab_t = emb_table.T
    tail_t = jnp.pad(
        lax.slice(emb_table, (TAIL_START, 0), (V, D)).T,
        ((0, 0), (0, 128 - (V - TAIL_START))))
    x2 = _make_sc_gather()(idx, tab_t, tail_t)
    out_t = _make_tc_silu_linear_t(O, 2048)(x2, W, b.reshape(O, 1))
    return out_t.T
